# Initial kernel scaffold; baseline (speedup 1.0000x reference)
import jax
import jax.numpy as jnp
from jax.experimental import pallas as pl


def kernel(img_lr, h_weight, l_weight):
    raise NotImplementedError("write your pallas kernel here")


# speedup vs baseline: 31.6317x; 31.6317x over previous
import functools

import jax
import jax.numpy as jnp
from jax import lax
from jax.experimental import pallas as pl
from jax.experimental.pallas import tpu as pltpu
from jax.experimental.pallas import tpu_sc as plsc

L = 128
H = 512
W = 512
NROWS = 6144
NW = 32
ROWS_PER_W = NROWS // NW

_ROTS = (
    ((0, 1), (0, 2), (1, 1), (0, 1, 2, 3)),
    ((1, 0), (2, 0), (1, -1), (2, 0, 3, 1)),
    ((0, -1), (0, -2), (-1, -1), (3, 2, 1, 0)),
    ((-1, 0), (-2, 0), (-1, 1), (1, 3, 0, 2)),
)


def _sc_body(img, hw, lw, out, rows_v, idxh_v, idxl_v, par_v, gh_v, gl_v,
             outb_v, sem):
    wid = lax.axis_index("s") * 2 + lax.axis_index("c")
    iota = lax.iota(jnp.int32, 16)

    def row_body(rl, carry):
        rglob = wid * ROWS_PER_W + rl
        ch = rglob // H
        i = rglob - ch * H
        base = ch * H

        for d in range(-2, 3):
            src = (base + jnp.clip(i + d, 0, H - 1)) * W
            pltpu.sync_copy(img.at[pl.ds(src, W)],
                            rows_v.at[pl.ds((d + 2) * W, W)])

        def idx_body(k, carry2):
            j0 = k * 16
            jv = j0 + iota
            jm2 = jnp.maximum(jv - 2, 0)
            jm1 = jnp.maximum(jv - 1, 0)
            jp1 = jnp.minimum(jv + 1, W - 1)
            jp2 = jnp.minimum(jv + 2, W - 1)
            colv = (jv, jp1, jp2, jm1, jm2)

            def ld(slot, cv):
                if cv == 0:
                    return rows_v[pl.ds(slot * W + j0, 16)]
                return plsc.load_gather(rows_v, [colv[cv] + slot * W])

            a = rows_v[pl.ds(2 * W + j0, 16)]
            av = a * (L * L)
            geom = (
                (2, 1, 2, 2, 3, 1),
                (3, 0, 4, 0, 3, 3),
                (2, 3, 2, 4, 1, 3),
                (1, 0, 0, 0, 1, 1),
            )
            for r, (bs, bc, hs, hc, ls, lc) in enumerate(geom):
                bb = av + ld(bs, bc) * L
                ih = bb + ld(hs, hc)
                il = bb + ld(ls, lc)
                off = r * 512 + j0
                idxh_v[pl.ds(off, 16)] = ih >> 1
                idxl_v[pl.ds(off, 16)] = il >> 1
                par_v[pl.ds(2 * off, 16)] = (ih << 2) & 4
                par_v[pl.ds(2 * off + 16, 16)] = (il << 2) & 4
            return carry2

        lax.fori_loop(0, 32, idx_body, 0, unroll=False)

        ch_ = pltpu.async_copy(hw.at[idxh_v], gh_v, sem)
        cl_ = pltpu.async_copy(lw.at[idxl_v], gl_v, sem)
        ch_.wait()
        cl_.wait()

        def comb_body(k, carry2):
            j0 = k * 16
            jv = j0 + iota
            acc = [jnp.zeros((16,), jnp.float32) for _ in range(4)]
            for r in range(4):
                perm = _ROTS[r][3]
                rowv = r * 512 + jv
                off = 2 * (r * 512 + j0)
                ph = par_v[pl.ds(off, 16)]
                plv = par_v[pl.ds(off + 16, 16)]
                for pos in range(4):
                    vh = plsc.load_gather(gh_v, [rowv, ph + perm[pos]])
                    vl = plsc.load_gather(gl_v, [rowv, plv + perm[pos]])
                    acc[pos] = acc[pos] + (vh + vl)
            jv2 = jv * 2
            plsc.store_scatter(outb_v, [jv2], acc[0] * 0.5)
            plsc.store_scatter(outb_v, [jv2 + 1], acc[1] * 0.5)
            plsc.store_scatter(outb_v, [jv2 + 1024], acc[2] * 0.5)
            plsc.store_scatter(outb_v, [jv2 + 1025], acc[3] * 0.5)
            return carry2

        lax.fori_loop(0, 32, comb_body, 0, unroll=False)

        pltpu.sync_copy(outb_v, out.at[pl.ds(rglob * 2048, 2048)])
        return carry

    lax.fori_loop(0, ROWS_PER_W, row_body, 0, unroll=False)


@functools.partial(
    pl.kernel,
    mesh=plsc.VectorSubcoreMesh(core_axis_name="c", subcore_axis_name="s"),
    out_type=jax.ShapeDtypeStruct((NROWS * 2 * W * 2,), jnp.float32),
    compiler_params=pltpu.CompilerParams(
        needs_layout_passes=False, use_tc_tiling_on_sc=False),
    scratch_types=[
        pltpu.VMEM((5 * W,), jnp.int32),
        pltpu.VMEM((4 * W,), jnp.int32),
        pltpu.VMEM((4 * W,), jnp.int32),
        pltpu.VMEM((8 * W,), jnp.int32),
        pltpu.VMEM((4 * W, 8), jnp.float32),
        pltpu.VMEM((4 * W, 8), jnp.float32),
        pltpu.VMEM((2 * 2 * W,), jnp.float32),
        pltpu.SemaphoreType.DMA,
    ],
)
def _hllut_sc(img, hw, lw, out, *refs):
    _sc_body(img, hw, lw, out, *refs)


def kernel(img_lr, h_weight, l_weight):
    B, C, _, _ = img_lr.shape
    img = img_lr.reshape(-1)
    hw = h_weight.reshape(-1, 8)
    lw = l_weight.reshape(-1, 8)
    out = _hllut_sc(img, hw, lw)
    return out.reshape(B, C, 2 * H, 2 * W)


# speedup vs baseline: 394.4214x; 12.4692x over previous
import functools

import jax
import jax.numpy as jnp
from jax import lax
from jax.experimental import pallas as pl
from jax.experimental.pallas import tpu as pltpu
from jax.experimental.pallas import tpu_sc as plsc

L = 128
H = 512
W = 512
NROWS = 6144
NW = 32
RPW = NROWS // NW

_PERMS = ((0, 1, 2, 3), (2, 0, 3, 1), (3, 2, 1, 0), (1, 3, 0, 2))
_GEOM = (
    (2, 1, 2, 2, 3, 1),
    (3, 0, 4, 0, 3, 3),
    (2, 3, 2, 4, 1, 3),
    (1, 0, 0, 0, 1, 1),
)

_MESH = plsc.VectorSubcoreMesh(core_axis_name="c", subcore_axis_name="s")
_CP = pltpu.CompilerParams(
    needs_layout_passes=False, use_tc_tiling_on_sc=False)


@functools.partial(
    pl.kernel,
    mesh=_MESH,
    out_type=(jax.ShapeDtypeStruct((8388608,), jnp.float32),
              jax.ShapeDtypeStruct((8388608,), jnp.float32)),
    compiler_params=_CP,
    scratch_types=[
        pltpu.VMEM((8192,), jnp.float32),
        pltpu.VMEM((8192,), jnp.float32),
    ],
)
def _reformat_sc(hw_n, lw_n, hw_o, lw_o, src_v, dst_v):
    wid = lax.axis_index("s") * 2 + lax.axis_index("c")
    iota = lax.iota(jnp.int32, 16)

    def table(src, dst):
        def chunk_body(ci, carry):
            t0 = wid * 512 + ci * 16
            for p in range(2):
                pltpu.sync_copy(src.at[pl.ds((p * 16384 + t0) * 256, 4096)],
                                src_v.at[pl.ds(p * 4096, 4096)])

            def grp_body(g, c2):
                tt = g // 8
                base_c = (g - tt * 8) * 16
                dstbase = (tt * 128 + base_c) * 4 + iota * 4
                for p in range(2):
                    for q in range(2):
                        v = src_v[pl.ds(p * 4096 + tt * 256 + q * 128
                                        + base_c, 16)]
                        plsc.store_scatter(dst_v, [dstbase + (2 * p + q)], v)
                return c2

            lax.fori_loop(0, 128, grp_body, 0, unroll=False)
            pltpu.sync_copy(dst_v, dst.at[pl.ds(t0 * 512, 8192)])
            return carry

        lax.fori_loop(0, 32, chunk_body, 0, unroll=False)

    table(hw_n, hw_o)
    table(lw_n, lw_o)


def _sc_body(img, hw, lw, out, rows_v, ih0, ih1, il0, il1, par_v,
             gh_v, gl_v, outb_v, rs0, rs1, gs0, gs1):
    wid = lax.axis_index("s") * 2 + lax.axis_index("c")
    iota = lax.iota(jnp.int32, 16)
    rsem = (rs0, rs1)
    gsem = (gs0, gs1)
    ih = (ih0, ih1)
    il = (il0, il1)

    def fire_rows(rl, s):
        rglob = wid * RPW + rl
        ch = rglob // H
        i = rglob - ch * H
        base = ch * H
        for d in range(-2, 3):
            src = (base + jnp.clip(i + d, 0, H - 1)) * W
            pltpu.async_copy(img.at[pl.ds(src, W)],
                             rows_v.at[pl.ds(s * 2560 + (d + 2) * W, W)],
                             rsem[s])

    def wait_rows(s):
        pltpu.make_async_copy(img.at[pl.ds(0, 2560)],
                              rows_v.at[pl.ds(s * 2560, 2560)],
                              rsem[s]).wait()

    def idx_compute(s):
        rb = s * 2560
        pb = s * 4096

        def idx_body(k, carry2):
            j0 = k * 16
            jv = j0 + iota
            colv = (jv, jnp.minimum(jv + 1, W - 1),
                    jnp.minimum(jv + 2, W - 1),
                    jnp.maximum(jv - 1, 0), jnp.maximum(jv - 2, 0))

            def ld(slot, cv):
                if cv == 0:
                    return rows_v[pl.ds(rb + slot * W + j0, 16)]
                return plsc.load_gather(rows_v, [colv[cv] + (rb + slot * W)])

            a = rows_v[pl.ds(rb + 2 * W + j0, 16)]
            av = a * (L * L)
            for r, (bs, bc, hs, hc, ls, lc) in enumerate(_GEOM):
                bb = av + ld(bs, bc) * L
                ihv = bb + ld(hs, hc)
                ilv = bb + ld(ls, lc)
                off = r * 512 + j0
                ih[s][pl.ds(off, 16)] = ihv >> 1
                il[s][pl.ds(off, 16)] = ilv >> 1
                par_v[pl.ds(pb + 2 * off, 16)] = (ihv << 2) & 4
                par_v[pl.ds(pb + 2 * off + 16, 16)] = (ilv << 2) & 4
            return carry2

        lax.fori_loop(0, 32, idx_body, 0, unroll=False)

    def fire_gath(s):
        pltpu.async_copy(hw.at[ih[s]], gh_v.at[pl.ds(s * 2048, 2048)],
                         gsem[s])
        pltpu.async_copy(lw.at[il[s]], gl_v.at[pl.ds(s * 2048, 2048)],
                         gsem[s])

    def wait_gath(s):
        pltpu.make_async_copy(hw.at[pl.ds(0, 2048)],
                              gh_v.at[pl.ds(s * 2048, 2048)],
                              gsem[s]).wait()
        pltpu.make_async_copy(lw.at[pl.ds(0, 2048)],
                              gl_v.at[pl.ds(s * 2048, 2048)],
                              gsem[s]).wait()

    def combine(rl, s):
        gb = s * 2048
        pb = s * 4096

        def comb_body(k, carry2):
            j0 = k * 16
            jv = j0 + iota
            acc = [jnp.zeros((16,), jnp.float32) for _ in range(4)]
            for r in range(4):
                perm = _PERMS[r]
                rowv = gb + r * 512 + jv
                off = pb + 2 * (r * 512 + j0)
                ph = par_v[pl.ds(off, 16)]
                plv = par_v[pl.ds(off + 16, 16)]
                for pos in range(4):
                    vh = plsc.load_gather(gh_v, [rowv, ph + perm[pos]])
                    vl = plsc.load_gather(gl_v, [rowv, plv + perm[pos]])
                    acc[pos] = acc[pos] + (vh + vl)
            jv2 = jv * 2
            plsc.store_scatter(outb_v, [jv2], acc[0] * 0.5)
            plsc.store_scatter(outb_v, [jv2 + 1], acc[1] * 0.5)
            plsc.store_scatter(outb_v, [jv2 + 1024], acc[2] * 0.5)
            plsc.store_scatter(outb_v, [jv2 + 1025], acc[3] * 0.5)
            return carry2

        lax.fori_loop(0, 32, comb_body, 0, unroll=False)
        rglob = wid * RPW + rl
        pltpu.sync_copy(outb_v, out.at[pl.ds(rglob * 2048, 2048)])

    fire_rows(0, 0)
    fire_rows(1, 1)
    wait_rows(0)
    idx_compute(0)
    fire_gath(0)

    def iter_body(k, carry):
        rA = 2 * k

        @pl.when(rA + 2 < RPW)
        def _():
            fire_rows(rA + 2, 0)

        wait_rows(1)
        idx_compute(1)
        fire_gath(1)

        wait_gath(0)
        combine(rA, 0)

        @pl.when(rA + 3 < RPW)
        def _():
            fire_rows(rA + 3, 1)

        @pl.when(rA + 2 < RPW)
        def _():
            wait_rows(0)
            idx_compute(0)
            fire_gath(0)

        wait_gath(1)
        combine(rA + 1, 1)
        return carry

    lax.fori_loop(0, RPW // 2, iter_body, 0, unroll=False)


@functools.partial(
    pl.kernel,
    mesh=_MESH,
    out_type=jax.ShapeDtypeStruct((NROWS * 2 * W * 2,), jnp.float32),
    compiler_params=_CP,
    scratch_types=[
        pltpu.VMEM((2 * 5 * W,), jnp.int32),
        pltpu.VMEM((4 * W,), jnp.int32),
        pltpu.VMEM((4 * W,), jnp.int32),
        pltpu.VMEM((4 * W,), jnp.int32),
        pltpu.VMEM((4 * W,), jnp.int32),
        pltpu.VMEM((2 * 8 * W,), jnp.int32),
        pltpu.VMEM((8 * W, 8), jnp.float32),
        pltpu.VMEM((8 * W, 8), jnp.float32),
        pltpu.VMEM((2 * 2 * W,), jnp.float32),
        pltpu.SemaphoreType.DMA,
        pltpu.SemaphoreType.DMA,
        pltpu.SemaphoreType.DMA,
        pltpu.SemaphoreType.DMA,
    ],
)
def _hllut_sc(img, hw, lw, out, *refs):
    _sc_body(img, hw, lw, out, *refs)


def _native_view(w):
    return w.reshape(16384, 128, 2, 2).transpose(2, 0, 3, 1).reshape(-1)


def kernel(img_lr, h_weight, l_weight):
    B, C, _, _ = img_lr.shape
    img = img_lr.reshape(-1)
    hw_lin, lw_lin = _reformat_sc(_native_view(h_weight),
                                  _native_view(l_weight))
    out = _hllut_sc(img, hw_lin.reshape(-1, 8), lw_lin.reshape(-1, 8))
    return out.reshape(B, C, 2 * H, 2 * W)


# speedup vs baseline: 438.8362x; 1.1126x over previous
import functools

import jax
import jax.numpy as jnp
from jax import lax
from jax.experimental import pallas as pl
from jax.experimental.pallas import tpu as pltpu
from jax.experimental.pallas import tpu_sc as plsc

L = 128
H = 512
W = 512
NROWS = 6144
NW = 32
RPW = NROWS // NW

_PERMS = ((0, 1, 2, 3), (2, 0, 3, 1), (3, 2, 1, 0), (1, 3, 0, 2))
_GEOM = (
    (2, 1, 2, 2, 3, 1),
    (3, 0, 4, 0, 3, 3),
    (2, 3, 2, 4, 1, 3),
    (1, 0, 0, 0, 1, 1),
)

_MESH = plsc.VectorSubcoreMesh(core_axis_name="c", subcore_axis_name="s")
_CP = pltpu.CompilerParams(
    needs_layout_passes=False, use_tc_tiling_on_sc=False)


@functools.partial(
    pl.kernel,
    mesh=_MESH,
    out_type=(jax.ShapeDtypeStruct((8388608,), jnp.float32),
              jax.ShapeDtypeStruct((8388608,), jnp.float32)),
    compiler_params=_CP,
    scratch_types=[
        pltpu.VMEM((2 * 8192,), jnp.float32),
        pltpu.VMEM((2 * 8192,), jnp.float32),
        pltpu.SemaphoreType.DMA,
        pltpu.SemaphoreType.DMA,
        pltpu.SemaphoreType.DMA,
        pltpu.SemaphoreType.DMA,
    ],
)
def _reformat_sc(hw_n, lw_n, hw_o, lw_o, src_v, dst_v, ss0, ss1, ds0, ds1):
    wid = lax.axis_index("s") * 2 + lax.axis_index("c")
    iota = lax.iota(jnp.int32, 16)
    ssem = (ss0, ss1)
    dsem = (ds0, ds1)

    def table(src, dst):
        def fire_src(ci, s):
            t0 = wid * 512 + ci * 16
            for p in range(2):
                pltpu.async_copy(
                    src.at[pl.ds((p * 16384 + t0) * 256, 4096)],
                    src_v.at[pl.ds(s * 8192 + p * 4096, 4096)], ssem[s])

        def wait_src(s):
            pltpu.make_async_copy(src.at[pl.ds(0, 8192)],
                                  src_v.at[pl.ds(s * 8192, 8192)],
                                  ssem[s]).wait()

        def wait_dst(s):
            pltpu.make_async_copy(src.at[pl.ds(0, 8192)],
                                  dst_v.at[pl.ds(s * 8192, 8192)],
                                  dsem[s]).wait()

        def interleave(ci, s):
            sb = s * 8192
            db = s * 8192

            def grp_body(g, c2):
                tt = g // 8
                base_c = (g - tt * 8) * 16
                dstbase = db + (tt * 128 + base_c) * 4 + iota * 4
                for p in range(2):
                    for q in range(2):
                        v = src_v[pl.ds(sb + p * 4096 + tt * 256 + q * 128
                                        + base_c, 16)]
                        plsc.store_scatter(dst_v, [dstbase + (2 * p + q)], v)
                return c2

            lax.fori_loop(0, 128, grp_body, 0, unroll=2)
            t0 = wid * 512 + ci * 16
            pltpu.async_copy(dst_v.at[pl.ds(db, 8192)],
                             dst.at[pl.ds(t0 * 512, 8192)], dsem[s])

        fire_src(0, 0)

        def chunk_body(k, carry):
            ciA = 2 * k
            fire_src(ciA + 1, 1)
            wait_src(0)

            @pl.when(k > 0)
            def _():
                wait_dst(0)

            interleave(ciA, 0)

            @pl.when(ciA + 2 < 32)
            def _():
                fire_src(ciA + 2, 0)

            wait_src(1)

            @pl.when(k > 0)
            def _():
                wait_dst(1)

            interleave(ciA + 1, 1)
            return carry

        lax.fori_loop(0, 16, chunk_body, 0, unroll=False)
        wait_dst(0)
        wait_dst(1)

    table(hw_n, hw_o)
    table(lw_n, lw_o)


def _sc_body(img, hw, lw, out, rows_v, ih0, ih1, il0, il1, par_v,
             gh_v, gl_v, outb_v, rs0, rs1, gs0, gs1, os0, os1):
    wid = lax.axis_index("s") * 2 + lax.axis_index("c")
    iota = lax.iota(jnp.int32, 16)
    rsem = (rs0, rs1)
    gsem = (gs0, gs1)
    osem = (os0, os1)
    ih = (ih0, ih1)
    il = (il0, il1)

    def fire_rows(rl, s):
        rglob = wid * RPW + rl
        ch = rglob // H
        i = rglob - ch * H
        base = ch * H
        for d in range(-2, 3):
            src = (base + jnp.clip(i + d, 0, H - 1)) * W
            pltpu.async_copy(img.at[pl.ds(src, W)],
                             rows_v.at[pl.ds(s * 2560 + (d + 2) * W, W)],
                             rsem[s])

    def wait_rows(s):
        pltpu.make_async_copy(img.at[pl.ds(0, 2560)],
                              rows_v.at[pl.ds(s * 2560, 2560)],
                              rsem[s]).wait()

    def idx_compute(s):
        rb = s * 2560
        pb = s * 4096

        def idx_body(k, carry2):
            j0 = k * 16
            jv = j0 + iota
            colv = (jv, jnp.minimum(jv + 1, W - 1),
                    jnp.minimum(jv + 2, W - 1),
                    jnp.maximum(jv - 1, 0), jnp.maximum(jv - 2, 0))

            def ld(slot, cv):
                if cv == 0:
                    return rows_v[pl.ds(rb + slot * W + j0, 16)]
                return plsc.load_gather(rows_v, [colv[cv] + (rb + slot * W)])

            a = rows_v[pl.ds(rb + 2 * W + j0, 16)]
            av = a * (L * L)
            for r, (bs, bc, hs, hc, ls, lc) in enumerate(_GEOM):
                bb = av + ld(bs, bc) * L
                ihv = bb + ld(hs, hc)
                ilv = bb + ld(ls, lc)
                off = r * 512 + j0
                ih[s][pl.ds(off, 16)] = ihv >> 1
                il[s][pl.ds(off, 16)] = ilv >> 1
                par_v[pl.ds(pb + 2 * off, 16)] = (ihv << 2) & 4
                par_v[pl.ds(pb + 2 * off + 16, 16)] = (ilv << 2) & 4
            return carry2

        lax.fori_loop(0, 32, idx_body, 0, unroll=2)

    def fire_gath(s):
        pltpu.async_copy(hw.at[ih[s]], gh_v.at[pl.ds(s * 2048, 2048)],
                         gsem[s])
        pltpu.async_copy(lw.at[il[s]], gl_v.at[pl.ds(s * 2048, 2048)],
                         gsem[s])

    def wait_gath(s):
        pltpu.make_async_copy(hw.at[pl.ds(0, 2048)],
                              gh_v.at[pl.ds(s * 2048, 2048)],
                              gsem[s]).wait()
        pltpu.make_async_copy(lw.at[pl.ds(0, 2048)],
                              gl_v.at[pl.ds(s * 2048, 2048)],
                              gsem[s]).wait()

    def combine(rl, s):
        gb = s * 2048
        pb = s * 4096
        ob = s * 2048

        @pl.when(rl >= 2)
        def _():
            pltpu.make_async_copy(out.at[pl.ds(0, 2048)],
                                  outb_v.at[pl.ds(ob, 2048)],
                                  osem[s]).wait()

        def comb_body(k, carry2):
            j0 = k * 16
            jv = j0 + iota
            acc = [jnp.zeros((16,), jnp.float32) for _ in range(4)]
            for r in range(4):
                perm = _PERMS[r]
                rowv = gb + r * 512 + jv
                off = pb + 2 * (r * 512 + j0)
                ph = par_v[pl.ds(off, 16)]
                plv = par_v[pl.ds(off + 16, 16)]
                for pos in range(4):
                    vh = plsc.load_gather(gh_v, [rowv, ph + perm[pos]])
                    vl = plsc.load_gather(gl_v, [rowv, plv + perm[pos]])
                    acc[pos] = acc[pos] + (vh + vl)
            jv2 = ob + jv * 2
            plsc.store_scatter(outb_v, [jv2], acc[0] * 0.5)
            plsc.store_scatter(outb_v, [jv2 + 1], acc[1] * 0.5)
            plsc.store_scatter(outb_v, [jv2 + 1024], acc[2] * 0.5)
            plsc.store_scatter(outb_v, [jv2 + 1025], acc[3] * 0.5)
            return carry2

        lax.fori_loop(0, 32, comb_body, 0, unroll=2)
        rglob = wid * RPW + rl
        pltpu.async_copy(outb_v.at[pl.ds(ob, 2048)],
                         out.at[pl.ds(rglob * 2048, 2048)], osem[s])

    fire_rows(0, 0)
    fire_rows(1, 1)
    wait_rows(0)
    idx_compute(0)
    fire_gath(0)

    def iter_body(k, carry):
        rA = 2 * k

        @pl.when(rA + 2 < RPW)
        def _():
            fire_rows(rA + 2, 0)

        wait_rows(1)
        idx_compute(1)
        fire_gath(1)

        wait_gath(0)
        combine(rA, 0)

        @pl.when(rA + 3 < RPW)
        def _():
            fire_rows(rA + 3, 1)

        @pl.when(rA + 2 < RPW)
        def _():
            wait_rows(0)
            idx_compute(0)
            fire_gath(0)

        wait_gath(1)
        combine(rA + 1, 1)
        return carry

    lax.fori_loop(0, RPW // 2, iter_body, 0, unroll=False)
    for s in range(2):
        pltpu.make_async_copy(out.at[pl.ds(0, 2048)],
                              outb_v.at[pl.ds(s * 2048, 2048)],
                              osem[s]).wait()


@functools.partial(
    pl.kernel,
    mesh=_MESH,
    out_type=jax.ShapeDtypeStruct((NROWS * 2 * W * 2,), jnp.float32),
    compiler_params=_CP,
    scratch_types=[
        pltpu.VMEM((2 * 5 * W,), jnp.int32),
        pltpu.VMEM((4 * W,), jnp.int32),
        pltpu.VMEM((4 * W,), jnp.int32),
        pltpu.VMEM((4 * W,), jnp.int32),
        pltpu.VMEM((4 * W,), jnp.int32),
        pltpu.VMEM((2 * 8 * W,), jnp.int32),
        pltpu.VMEM((8 * W, 8), jnp.float32),
        pltpu.VMEM((8 * W, 8), jnp.float32),
        pltpu.VMEM((2 * 2 * 2 * W,), jnp.float32),
        pltpu.SemaphoreType.DMA,
        pltpu.SemaphoreType.DMA,
        pltpu.SemaphoreType.DMA,
        pltpu.SemaphoreType.DMA,
        pltpu.SemaphoreType.DMA,
        pltpu.SemaphoreType.DMA,
    ],
)
def _hllut_sc(img, hw, lw, out, *refs):
    _sc_body(img, hw, lw, out, *refs)


def _native_view(w):
    return w.reshape(16384, 128, 2, 2).transpose(2, 0, 3, 1).reshape(-1)


def kernel(img_lr, h_weight, l_weight):
    B, C, _, _ = img_lr.shape
    img = img_lr.reshape(-1)
    hw_lin, lw_lin = _reformat_sc(_native_view(h_weight),
                                  _native_view(l_weight))
    out = _hllut_sc(img, hw_lin.reshape(-1, 8), lw_lin.reshape(-1, 8))
    return out.reshape(B, C, 2 * H, 2 * W)
